# baseline (speedup 1.0000x reference)
import jax
import jax.numpy as jnp
from jax import lax
from jax.experimental import pallas as pl
from jax.experimental.pallas import tpu as pltpu
from jax.experimental.pallas import tpu_sc as plsc

_N = 16384
_D = 4
_L = 16
_NC = 2
_NW = 32
_WPR = _NW // _D
_CHUNK = _N // _WPR


def _sc_body(w1_hbm, w2_hbm, out_hbm, w1_v, w2_v, out_v, sem):
    wid = lax.axis_index("s") * _NC + lax.axis_index("c")
    row = jnp.right_shift(wid, 3)
    cp1 = pltpu.async_copy(w1_hbm, w1_v, sem)
    cp2 = pltpu.async_copy(w2_hbm, w2_v, sem)
    cp1.wait()
    cp2.wait()
    zero = jnp.zeros((_L,), jnp.int32)
    ridx = jnp.full((_L,), row, jnp.int32)
    v1 = plsc.load_gather(w1_v, [zero, ridx])
    v2 = plsc.load_gather(w2_v, [zero, ridx])
    vec = jnp.sign(v1) * jnp.sign(v2)
    for i in range(_CHUNK // _L):
        out_v[pl.ds(i * _L, _L)] = vec
    pltpu.sync_copy(out_v, out_hbm.at[pl.ds(wid * _CHUNK, _CHUNK)])


@jax.jit
def _run(w1, w2):
    mesh = plsc.VectorSubcoreMesh(core_axis_name="c", subcore_axis_name="s")
    out_t = pl.kernel(
        _sc_body,
        out_type=jax.ShapeDtypeStruct((_D * _N,), jnp.float32),
        mesh=mesh,
        compiler_params=pltpu.CompilerParams(needs_layout_passes=False),
        scratch_types=[
            pltpu.VMEM((1, _D), jnp.float32),
            pltpu.VMEM((1, _D), jnp.float32),
            pltpu.VMEM((_CHUNK,), jnp.float32),
            pltpu.SemaphoreType.DMA,
        ],
    )(w1, w2)
    return out_t.reshape(_D, _N).T


def kernel(input, data1_weight, data2_weight):
    del input
    return _run(data1_weight, data2_weight)


# speedup vs baseline: 1.1005x; 1.1005x over previous
import jax
import jax.numpy as jnp
from jax import lax
from jax.experimental import pallas as pl
from jax.experimental.pallas import tpu as pltpu
from jax.experimental.pallas import tpu_sc as plsc

_N = 16384
_D = 4
_L = 16
_NC = 2
_NW = 32
_WPR = _NW // _D
_CHUNK = _N // _WPR


def _sc_body(w1_hbm, w2_hbm, out_hbm, w1_v, w2_v, out_v, sem):
    wid = lax.axis_index("s") * _NC + lax.axis_index("c")
    row = jnp.right_shift(wid, 3)
    col0 = jnp.bitwise_and(wid, _WPR - 1) * _CHUNK
    cp1 = pltpu.async_copy(w1_hbm, w1_v, sem)
    cp2 = pltpu.async_copy(w2_hbm, w2_v, sem)
    cp1.wait()
    cp2.wait()
    zero = jnp.zeros((_L,), jnp.int32)
    ridx = jnp.full((_L,), row, jnp.int32)
    v1 = plsc.load_gather(w1_v, [zero, ridx])
    v2 = plsc.load_gather(w2_v, [zero, ridx])
    vec = jnp.sign(v1) * jnp.sign(v2)
    for i in range(_CHUNK // _L):
        out_v[pl.ds(i * _L, _L)] = vec
    pltpu.sync_copy(out_v, out_hbm.at[row, pl.ds(col0, _CHUNK)])


@jax.jit
def _run(w1, w2):
    mesh = plsc.VectorSubcoreMesh(core_axis_name="c", subcore_axis_name="s")
    out_t = pl.kernel(
        _sc_body,
        out_type=jax.ShapeDtypeStruct((_D, _N), jnp.float32),
        mesh=mesh,
        compiler_params=pltpu.CompilerParams(needs_layout_passes=False),
        scratch_types=[
            pltpu.VMEM((1, _D), jnp.float32),
            pltpu.VMEM((1, _D), jnp.float32),
            pltpu.VMEM((_CHUNK,), jnp.float32),
            pltpu.SemaphoreType.DMA,
        ],
    )(w1, w2)
    return out_t.T


def kernel(input, data1_weight, data2_weight):
    del input
    return _run(data1_weight, data2_weight)
